# Initial kernel scaffold; baseline (speedup 1.0000x reference)
import functools

import jax
import jax.numpy as jnp
from jax import lax
from jax.experimental import pallas as pl
from jax.experimental.pallas import tpu as pltpu
from jax.experimental.pallas import tpu_sc as plsc

M_ROWS = 250000
D = 128
N = M_ROWS * D
NC = 2
NS = 16
NW = NC * NS
SHARD = N // NW
CHUNK = 8000
NCHUNK = SHARD // CHUNK
GROUPS = CHUNK // 64
WORDS = CHUNK // 4
SRCW = CHUNK + 8

_mesh = plsc.VectorSubcoreMesh(core_axis_name="c", subcore_axis_name="s")


def _wid():
    return lax.axis_index("s") * NC + lax.axis_index("c")


def _bytesums(v):
    u = v + (v << 8)
    return u + (u << 16)


@functools.partial(
    pl.kernel,
    out_type=jax.ShapeDtypeStruct((NW * 128,), jnp.int32),
    mesh=_mesh,
    scratch_types=[
        pltpu.VMEM((WORDS,), jnp.int32),
        pltpu.VMEM((128,), jnp.int32),
    ],
)
def _count_kernel(mw_hbm, counts_hbm, mbuf, cbuf):
    w = _wid()
    zeros = jnp.zeros((16,), jnp.int32)
    lane = lax.iota(jnp.int32, 16)
    m0 = lane == 0
    for j in range(8):
        cbuf[pl.ds(j * 16, 16)] = zeros

    def chunk_body(i, _):
        pltpu.sync_copy(mw_hbm.at[pl.ds(w * (SHARD // 4) + i * WORDS, WORDS)],
                        mbuf)

        def g_body(g, acc):
            v = mbuf[pl.ds(g * 16, 16)]
            return acc + lax.shift_right_logical(_bytesums(v), 24)

        acc = lax.fori_loop(0, GROUPS, g_body, zeros)
        cnt = jnp.sum(acc)
        plsc.store_scatter(cbuf, [zeros + i], zeros + cnt, mask=m0)
        return 0

    lax.fori_loop(0, NCHUNK, chunk_body, 0)
    pltpu.sync_copy(cbuf, counts_hbm.at[pl.ds(w * 128, 128)])


@functools.partial(
    pl.kernel,
    out_type=jax.ShapeDtypeStruct((N,), jnp.float32),
    mesh=_mesh,
    scratch_types=[
        pltpu.VMEM((WORDS,), jnp.int32),
        pltpu.VMEM((CHUNK,), jnp.float32),
        pltpu.VMEM((SRCW,), jnp.float32),
        pltpu.VMEM((NW * 128,), jnp.int32),
    ],
)
def _scatter_kernel(mw_hbm, x_hbm, src_hbm, counts_hbm, out_hbm,
                    mbuf, obuf, sbuf, cbuf):
    w = _wid()
    zeros = jnp.zeros((16,), jnp.int32)
    lane = lax.iota(jnp.int32, 16)
    lane4 = lane * 4

    pltpu.sync_copy(counts_hbm, cbuf)

    def sb_body(j, acc):
        return acc + cbuf[pl.ds(j * 16, 16)]

    shard_base = jnp.sum(lax.fori_loop(0, w * 8, sb_body, zeros))

    def chunk_body(i, carry):
        start = w * SHARD + i * CHUNK
        pltpu.sync_copy(mw_hbm.at[pl.ds(w * (SHARD // 4) + i * WORDS, WORDS)],
                        mbuf)
        pltpu.sync_copy(x_hbm.at[pl.ds(start, CHUNK)], obuf)
        base8 = jnp.minimum(carry & -8, N - SRCW)
        pltpu.sync_copy(src_hbm.at[pl.ds(base8, SRCW)], sbuf)

        def g_body(g, cl):
            v = mbuf[pl.ds(g * 16, 16)]
            s = _bytesums(v)
            t = lax.shift_right_logical(s, 24)
            ex = plsc.cumsum(t) - t
            basev = ex + (cl - 1)
            pos0 = lane4 + g * 64
            for k in range(4):
                mk = (lax.shift_right_logical(v, 8 * k) & 1) == 1
                ck = lax.shift_right_logical(s, 8 * k) & 0xFF
                idx = jnp.maximum(basev + ck, 0)
                gk = plsc.load_gather(sbuf, [idx], mask=mk)
                plsc.store_scatter(obuf, [pos0 + k], gk, mask=mk)
            return cl + jnp.sum(t)

        cl_end = lax.fori_loop(0, GROUPS, g_body, carry - base8)
        pltpu.sync_copy(obuf, out_hbm.at[pl.ds(start, CHUNK)])
        return base8 + cl_end

    lax.fori_loop(0, NCHUNK, chunk_body, shard_base)


def kernel(x, mask, src):
    xf = x.reshape(-1)
    sf = src.reshape(-1)
    mw = lax.bitcast_convert_type(
        mask.astype(jnp.uint8).reshape(N // 4, 4), jnp.int32)
    counts = _count_kernel(mw)
    outf = _scatter_kernel(mw, xf, sf, counts)
    return outf.reshape(x.shape)


# speedup vs baseline: 1.1451x; 1.1451x over previous
import functools

import jax
import jax.numpy as jnp
from jax import lax
from jax.experimental import pallas as pl
from jax.experimental.pallas import tpu as pltpu
from jax.experimental.pallas import tpu_sc as plsc

M_ROWS = 250000
D = 128
N = M_ROWS * D
NC = 2
NS = 16
NW = NC * NS
SHARD = N // NW
CHUNK = 8000
NCHUNK = SHARD // CHUNK
GROUPS = CHUNK // 64
WORDS = CHUNK // 4
SRCW = CHUNK + 8

_mesh = plsc.VectorSubcoreMesh(core_axis_name="c", subcore_axis_name="s")
_params = pltpu.CompilerParams(needs_layout_passes=False)


def _wid():
    return lax.axis_index("s") * NC + lax.axis_index("c")


def _bytesums(v):
    u = v + (v << 8)
    return u + (u << 16)


@functools.partial(
    pl.kernel,
    out_type=jax.ShapeDtypeStruct((NW * 128,), jnp.int32),
    mesh=_mesh,
    compiler_params=_params,
    scratch_types=[
        pltpu.VMEM((WORDS,), jnp.int32),
        pltpu.VMEM((128,), jnp.int32),
    ],
)
def _count_kernel(mw_hbm, counts_hbm, mbuf, cbuf):
    w = _wid()
    zeros = jnp.zeros((16,), jnp.int32)
    lane = lax.iota(jnp.int32, 16)
    m0 = lane == 0
    for j in range(8):
        cbuf[pl.ds(j * 16, 16)] = zeros

    def chunk_body(i, _):
        pltpu.sync_copy(mw_hbm.at[pl.ds(w * (SHARD // 4) + i * WORDS, WORDS)],
                        mbuf)

        def g_body(g, acc):
            v = mbuf[pl.ds(g * 16, 16)]
            return acc + lax.shift_right_logical(_bytesums(v), 24)

        acc = lax.fori_loop(0, GROUPS, g_body, zeros)
        cnt = jnp.sum(acc)
        plsc.store_scatter(cbuf, [zeros + i], zeros + cnt, mask=m0)
        return 0

    lax.fori_loop(0, NCHUNK, chunk_body, 0)
    pltpu.sync_copy(cbuf, counts_hbm.at[pl.ds(w * 128, 128)])


@functools.partial(
    pl.kernel,
    out_type=jax.ShapeDtypeStruct((N,), jnp.float32),
    mesh=_mesh,
    compiler_params=_params,
    scratch_types=[
        pltpu.VMEM((WORDS,), jnp.int32),
        pltpu.VMEM((CHUNK,), jnp.float32),
        pltpu.VMEM((SRCW,), jnp.float32),
        pltpu.VMEM((NW * 128,), jnp.int32),
    ],
)
def _scatter_kernel(mw_hbm, x_hbm, src_hbm, counts_hbm, out_hbm,
                    mbuf, obuf, sbuf, cbuf):
    w = _wid()
    zeros = jnp.zeros((16,), jnp.int32)
    lane = lax.iota(jnp.int32, 16)
    lane4 = lane * 4

    pltpu.sync_copy(counts_hbm, cbuf)

    def sb_body(j, acc):
        return acc + cbuf[pl.ds(j * 16, 16)]

    shard_base = jnp.sum(lax.fori_loop(0, w * 8, sb_body, zeros))

    def chunk_body(i, carry):
        start = w * SHARD + i * CHUNK
        pltpu.sync_copy(mw_hbm.at[pl.ds(w * (SHARD // 4) + i * WORDS, WORDS)],
                        mbuf)
        pltpu.sync_copy(x_hbm.at[pl.ds(start, CHUNK)], obuf)
        base8 = pl.multiple_of(jnp.minimum(carry & -8, N - SRCW), 8)
        pltpu.sync_copy(src_hbm.at[pl.ds(base8, SRCW)], sbuf)

        def g_body(g, cl):
            v = mbuf[pl.ds(g * 16, 16)]
            s = _bytesums(v)
            t = lax.shift_right_logical(s, 24)
            ex = plsc.cumsum(t) - t
            basev = ex + (cl - 1)
            pos0 = lane4 + g * 64
            for k in range(4):
                mk = (lax.shift_right_logical(v, 8 * k) & 1) == 1
                ck = lax.shift_right_logical(s, 8 * k) & 0xFF
                idx = jnp.maximum(basev + ck, 0)
                gk = plsc.load_gather(sbuf, [idx], mask=mk)
                plsc.store_scatter(obuf, [pos0 + k], gk, mask=mk)
            return cl + jnp.sum(t)

        cl_end = lax.fori_loop(0, GROUPS, g_body, carry - base8)
        pltpu.sync_copy(obuf, out_hbm.at[pl.ds(start, CHUNK)])
        return base8 + cl_end

    lax.fori_loop(0, NCHUNK, chunk_body, shard_base)


def kernel(x, mask, src):
    xf = x.reshape(-1)
    sf = src.reshape(-1)
    mw = lax.bitcast_convert_type(
        mask.astype(jnp.uint8).reshape(N // 4, 4), jnp.int32)
    counts = _count_kernel(mw)
    outf = _scatter_kernel(mw, xf, sf, counts)
    return outf.reshape(x.shape)


# speedup vs baseline: 1.2615x; 1.1017x over previous
import functools

import jax
import jax.numpy as jnp
from jax import lax
from jax.experimental import pallas as pl
from jax.experimental.pallas import tpu as pltpu
from jax.experimental.pallas import tpu_sc as plsc

M_ROWS = 250000
D = 128
N = M_ROWS * D
NC = 2
NS = 16
NW = NC * NS
SHARD = N // NW
CHUNK = 8000
NCHUNK = SHARD // CHUNK
GROUPS = CHUNK // 64
WORDS = CHUNK // 4
SRCW = CHUNK + 8

_mesh = plsc.VectorSubcoreMesh(core_axis_name="c", subcore_axis_name="s")
_params = pltpu.CompilerParams(needs_layout_passes=False)


def _wid():
    return lax.axis_index("s") * NC + lax.axis_index("c")


def _bytesums(v):
    u = v + (v << 8)
    return u + (u << 16)


@functools.partial(
    pl.kernel,
    out_type=jax.ShapeDtypeStruct((NW * 128,), jnp.int32),
    mesh=_mesh,
    compiler_params=_params,
    scratch_types=[
        pltpu.VMEM((WORDS,), jnp.int32),
        pltpu.VMEM((128,), jnp.int32),
    ],
)
def _count_kernel(mw_hbm, counts_hbm, mbuf, cbuf):
    w = _wid()
    zeros = jnp.zeros((16,), jnp.int32)
    lane = lax.iota(jnp.int32, 16)
    m0 = lane == 0
    for j in range(8):
        cbuf[pl.ds(j * 16, 16)] = zeros

    def chunk_body(i, _):
        pltpu.sync_copy(mw_hbm.at[pl.ds(w * (SHARD // 4) + i * WORDS, WORDS)],
                        mbuf)

        def g_body(g, acc):
            v = mbuf[pl.ds(g * 16, 16)]
            return acc + lax.shift_right_logical(_bytesums(v), 24)

        acc = lax.fori_loop(0, GROUPS, g_body, zeros)
        cnt = jnp.sum(acc)
        plsc.store_scatter(cbuf, [zeros + i], zeros + cnt, mask=m0)
        return 0

    lax.fori_loop(0, NCHUNK, chunk_body, 0)
    pltpu.sync_copy(cbuf, counts_hbm.at[pl.ds(w * 128, 128)])


NSLOT = 3
NTRIPLE = 41


@functools.partial(
    pl.kernel,
    out_type=jax.ShapeDtypeStruct((N,), jnp.float32),
    mesh=_mesh,
    compiler_params=_params,
    scratch_types=(
        [pltpu.VMEM((WORDS,), jnp.int32) for _ in range(NSLOT)]
        + [pltpu.VMEM((CHUNK,), jnp.float32) for _ in range(NSLOT)]
        + [pltpu.VMEM((SRCW,), jnp.float32) for _ in range(NSLOT)]
        + [
            pltpu.VMEM((NW * 128,), jnp.int32),
            pltpu.VMEM((144,), jnp.int32),
            pltpu.VMEM((WORDS,), jnp.int32),
            pltpu.VMEM((144,), jnp.int32),
        ]
        + [pltpu.SemaphoreType.DMA for _ in range(3 * NSLOT)]
    ),
)
def _scatter_kernel(mw_hbm, x_hbm, src_hbm, counts_hbm, out_hbm,
                    mb0, mb1, mb2, ob0, ob1, ob2, sb0, sb1, sb2,
                    cbuf, bbuf, wbuf, tbuf,
                    si0, si1, si2, sx0, sx1, sx2, so0, so1, so2):
    mbufs = (mb0, mb1, mb2)
    obufs = (ob0, ob1, ob2)
    sbufs = (sb0, sb1, sb2)
    sem_in = (si0, si1, si2)
    sem_x = (sx0, sx1, sx2)
    sem_out = (so0, so1, so2)
    w = _wid()
    zeros = jnp.zeros((16,), jnp.int32)
    lane = lax.iota(jnp.int32, 16)
    lane4 = lane * 4
    m15 = lane == 15

    pltpu.sync_copy(counts_hbm, cbuf)

    def sb_body(j, acc):
        return acc + cbuf[pl.ds(j * 16, 16)]

    shard_base = jnp.sum(lax.fori_loop(0, w * 8, sb_body, zeros))

    def bb_body(j, carry_s):
        cvec = cbuf[pl.ds((w * 8 + j) * 16, 16)]
        bbuf[pl.ds(j * 16, 16)] = plsc.cumsum(cvec) - cvec + carry_s
        return carry_s + jnp.sum(cvec)

    lax.fori_loop(0, 8, bb_body, shard_base)

    def chunk_base(i):
        return bbuf[pl.ds(i, 16)][0]

    def win_base(i):
        return pl.multiple_of(
            jnp.minimum(chunk_base(i) & -8, N - SRCW), 8)

    def in_copies(i, u):
        start = w * SHARD + i * CHUNK
        m = pltpu.make_async_copy(
            mw_hbm.at[pl.ds(w * (SHARD // 4) + i * WORDS, WORDS)],
            mbufs[u], sem_in[u])
        s = pltpu.make_async_copy(
            src_hbm.at[pl.ds(win_base(i), SRCW)], sbufs[u], sem_in[u])
        xc = pltpu.make_async_copy(
            x_hbm.at[pl.ds(start, CHUNK)], obufs[u], sem_x[u])
        return m, s, xc

    def out_copy(i, u):
        start = w * SHARD + i * CHUNK
        return pltpu.make_async_copy(
            obufs[u], out_hbm.at[pl.ds(start, CHUNK)], sem_out[u])

    def do_chunk(i, u, prefetch):
        mbuf, obuf, sbuf = mbufs[u], obufs[u], sbufs[u]
        for c in in_copies(i, u):
            c.wait()

        base_s = chunk_base(i)
        base8 = jnp.minimum(base_s & -8, N - SRCW)

        @plsc.parallel_loop(0, GROUPS)
        def _(g):
            v = mbuf[pl.ds(g * 16, 16)]
            s = _bytesums(v)
            t = lax.shift_right_logical(s, 24)
            tc = plsc.cumsum(t)
            wbuf[pl.ds(g * 16, 16)] = tc - t
            plsc.store_scatter(tbuf, [zeros + g], tc, mask=m15)

        def gb_body(j, carry_s):
            gvec = tbuf[pl.ds(j * 16, 16)]
            tbuf[pl.ds(j * 16, 16)] = plsc.cumsum(gvec) - gvec + carry_s
            return carry_s + jnp.sum(gvec)

        lax.fori_loop(0, 8, gb_body, base_s - base8 - 1)

        @plsc.parallel_loop(0, GROUPS)
        def _(g):
            v = mbuf[pl.ds(g * 16, 16)]
            s = _bytesums(v)
            basev = wbuf[pl.ds(g * 16, 16)] + tbuf[pl.ds(g, 16)][0]
            pos0 = lane4 + g * 64
            for k in range(4):
                mk = (lax.shift_right_logical(v, 8 * k) & 1) == 1
                ck = lax.shift_right_logical(s, 8 * k) & 0xFF
                idx = jnp.maximum(basev + ck, 0)
                gk = plsc.load_gather(sbuf, [idx], mask=mk)
                plsc.store_scatter(obuf, [pos0 + k], gk, mask=mk)

        out_copy(i, u).start()

        if prefetch:
            nu = (u + 2) % NSLOT

            @pl.when(i >= 1)
            def _():
                out_copy(i - 1, nu).wait()

            for c in in_copies(i + 2, nu):
                c.start()

    for c in in_copies(0, 0) + in_copies(1, 1):
        c.start()

    def triple_body(p, _):
        for u in range(NSLOT):
            do_chunk(3 * p + u, u, prefetch=True)
        return 0

    lax.fori_loop(0, NTRIPLE, triple_body, 0)
    do_chunk(NCHUNK - 2, 0, prefetch=False)
    do_chunk(NCHUNK - 1, 1, prefetch=False)
    out_copy(NCHUNK - 3, 2).wait()
    out_copy(NCHUNK - 2, 0).wait()
    out_copy(NCHUNK - 1, 1).wait()


def kernel(x, mask, src):
    xf = x.reshape(-1)
    sf = src.reshape(-1)
    mw = lax.bitcast_convert_type(
        mask.astype(jnp.uint8).reshape(N // 4, 4), jnp.int32)
    counts = _count_kernel(mw)
    outf = _scatter_kernel(mw, xf, sf, counts)
    return outf.reshape(x.shape)


# speedup vs baseline: 8.5454x; 6.7739x over previous
import functools

import jax
import jax.numpy as jnp
from jax import lax
from jax.experimental import pallas as pl
from jax.experimental.pallas import tpu as pltpu
from jax.experimental.pallas import tpu_sc as plsc

M_ROWS = 250000
D = 128
N = M_ROWS * D
NC = 2
NS = 16
NW = NC * NS
CHUNK = 10000
NCHUNK = 100
GROUPS = CHUNK // 16
GVECS = 40
SRCW = CHUNK + 8
CROW = 112
NSLOT = 3

_mesh = plsc.VectorSubcoreMesh(core_axis_name="c", subcore_axis_name="s")
_params = pltpu.CompilerParams(needs_layout_passes=False)


def _wid():
    return lax.axis_index("s") * NC + lax.axis_index("c")


@functools.partial(
    pl.kernel,
    out_type=jax.ShapeDtypeStruct((NW * CROW,), jnp.int32),
    mesh=_mesh,
    compiler_params=_params,
    scratch_types=[
        pltpu.VMEM((CHUNK,), jnp.float32),
        pltpu.VMEM((CHUNK,), jnp.float32),
        pltpu.VMEM((CROW,), jnp.int32),
        pltpu.SemaphoreType.DMA,
        pltpu.SemaphoreType.DMA,
    ],
)
def _count_kernel(mk_hbm, counts_hbm, mb0, mb1, cbuf, sm0, sm1):
    w = _wid()
    mbufs = (mb0, mb1)
    sems = (sm0, sm1)
    zeros = jnp.zeros((16,), jnp.int32)
    fzeros = jnp.zeros((16,), jnp.float32)
    lane = lax.iota(jnp.int32, 16)
    m0 = lane == 0
    for j in range(CROW // 16):
        cbuf[pl.ds(j * 16, 16)] = zeros

    def mask_copy(i, u):
        return pltpu.make_async_copy(
            mk_hbm.at[pl.ds((w * NCHUNK + i) * CHUNK, CHUNK)],
            mbufs[u], sems[u])

    mask_copy(0, 0).start()

    def pair_body(p, _):
        for u in range(2):
            i = 2 * p + u
            mask_copy(i, u).wait()

            @pl.when(i + 1 < NCHUNK)
            def _():
                mask_copy(i + 1, 1 - u).start()

            def g_body(g, acc):
                return acc + mbufs[u][pl.ds(g * 16, 16)]

            acc = lax.fori_loop(0, GROUPS, g_body, fzeros)
            cnt = jnp.sum(acc).astype(jnp.int32)
            plsc.store_scatter(cbuf, [zeros + i], zeros + cnt, mask=m0)

        return 0

    lax.fori_loop(0, NCHUNK // 2, pair_body, 0)
    pltpu.sync_copy(cbuf, counts_hbm.at[pl.ds(w * CROW, CROW)])


@functools.partial(
    pl.kernel,
    out_type=jax.ShapeDtypeStruct((N,), jnp.float32),
    mesh=_mesh,
    compiler_params=_params,
    scratch_types=(
        [pltpu.VMEM((CHUNK,), jnp.float32) for _ in range(NSLOT)]
        + [pltpu.VMEM((CHUNK,), jnp.float32) for _ in range(NSLOT)]
        + [pltpu.VMEM((SRCW,), jnp.float32) for _ in range(NSLOT)]
        + [
            pltpu.VMEM((NW * CROW,), jnp.int32),
            pltpu.VMEM((128,), jnp.int32),
            pltpu.VMEM((CHUNK,), jnp.int32),
            pltpu.VMEM((16 * GVECS + 16,), jnp.int32),
        ]
        + [pltpu.SemaphoreType.DMA for _ in range(3 * NSLOT)]
    ),
)
def _scatter_kernel(mk_hbm, x_hbm, src_hbm, counts_hbm, out_hbm,
                    mb0, mb1, mb2, ob0, ob1, ob2, sb0, sb1, sb2,
                    cbuf, bbuf, wbuf, tbuf,
                    si0, si1, si2, sx0, sx1, sx2, so0, so1, so2):
    mbufs = (mb0, mb1, mb2)
    obufs = (ob0, ob1, ob2)
    sbufs = (sb0, sb1, sb2)
    sem_in = (si0, si1, si2)
    sem_x = (sx0, sx1, sx2)
    sem_out = (so0, so1, so2)
    w = _wid()
    zeros = jnp.zeros((16,), jnp.int32)
    lane = lax.iota(jnp.int32, 16)
    m15 = lane == 15

    pltpu.sync_copy(counts_hbm, cbuf)

    def sb_body(j, acc):
        return acc + cbuf[pl.ds(j * 16, 16)]

    shard_base = jnp.sum(
        lax.fori_loop(0, w * (CROW // 16), sb_body, zeros))

    def bb_body(j, carry_s):
        cvec = cbuf[pl.ds((w * (CROW // 16) + j) * 16, 16)]
        bbuf[pl.ds(j * 16, 16)] = plsc.cumsum(cvec) - cvec + carry_s
        return carry_s + jnp.sum(cvec)

    lax.fori_loop(0, CROW // 16, bb_body, shard_base)

    def chunk_base(i):
        return bbuf[pl.ds(i, 16)][0]

    def win_base(i):
        return pl.multiple_of(
            jnp.minimum(chunk_base(i) & -8, N - SRCW), 8)

    def in_copies(i, u):
        start = (w * NCHUNK + i) * CHUNK
        m = pltpu.make_async_copy(
            mk_hbm.at[pl.ds(start, CHUNK)], mbufs[u], sem_in[u])
        s = pltpu.make_async_copy(
            src_hbm.at[pl.ds(win_base(i), SRCW)], sbufs[u], sem_in[u])
        xc = pltpu.make_async_copy(
            x_hbm.at[pl.ds(start, CHUNK)], obufs[u], sem_x[u])
        return m, s, xc

    def out_copy(i, u):
        start = (w * NCHUNK + i) * CHUNK
        return pltpu.make_async_copy(
            obufs[u], out_hbm.at[pl.ds(start, CHUNK)], sem_out[u])

    def do_chunk(i, u):
        mbuf, obuf, sbuf = mbufs[u], obufs[u], sbufs[u]
        for c in in_copies(i, u):
            c.wait()

        nu = (u + 2) % NSLOT

        @pl.when(i + 2 < NCHUNK)
        def _():
            @pl.when(i >= 1)
            def _():
                out_copy(i - 1, nu).wait()

            for c in in_copies(i + 2, nu):
                c.start()

        base_s = chunk_base(i)
        base8 = jnp.minimum(base_s & -8, N - SRCW)

        @plsc.parallel_loop(0, GROUPS)
        def _(g):
            m = mbuf[pl.ds(g * 16, 16)]
            tc = plsc.cumsum(m).astype(jnp.int32)
            wbuf[pl.ds(g * 16, 16)] = tc - m.astype(jnp.int32)
            plsc.store_scatter(tbuf, [zeros + g], tc, mask=m15)

        def gb_body(j, carry_s):
            gvec = tbuf[pl.ds(j * 16, 16)]
            tbuf[pl.ds(j * 16, 16)] = plsc.cumsum(gvec) - gvec + carry_s
            return carry_s + jnp.sum(gvec)

        lax.fori_loop(0, GVECS, gb_body, base_s - base8)

        @plsc.parallel_loop(0, GROUPS)
        def _(g):
            mk = mbuf[pl.ds(g * 16, 16)] != 0.0
            idx = wbuf[pl.ds(g * 16, 16)] + tbuf[pl.ds(g, 16)][0]
            gk = plsc.load_gather(sbuf, [idx], mask=mk)
            plsc.store_scatter(obuf, [lane + g * 16], gk, mask=mk)

        out_copy(i, u).start()

    for c in in_copies(0, 0) + in_copies(1, 1):
        c.start()

    def triple_body(p, _):
        for u in range(NSLOT):
            i = 3 * p + u

            @pl.when(i < NCHUNK)
            def _():
                do_chunk(i, u)

        return 0

    lax.fori_loop(0, (NCHUNK + NSLOT - 1) // NSLOT, triple_body, 0)
    for u in range(NSLOT):
        out_copy(0, u).wait()


def kernel(x, mask, src):
    xf = x.reshape(-1)
    sf = src.reshape(-1)
    mf = mask.astype(jnp.float32).reshape(-1)
    counts = _count_kernel(mf)
    outf = _scatter_kernel(mf, xf, sf, counts)
    return outf.reshape(x.shape)


# speedup vs baseline: 13.2346x; 1.5487x over previous
import functools

import jax
import jax.numpy as jnp
from jax import lax
from jax.experimental import pallas as pl
from jax.experimental.pallas import tpu as pltpu
from jax.experimental.pallas import tpu_sc as plsc

M_ROWS = 250000
D = 128
N = M_ROWS * D
NC = 2
NS = 16
NW = NC * NS
CHUNK = 12800
NCHUNK_G = N // CHUNK
GROUPS = CHUNK // 16
WGROUPS = CHUNK // 64
MAXCH = 79
SRCW = CHUNK + 8
SPAD = SRCW + 16
CROW = 80
NSLOT = 3

_mesh = plsc.VectorSubcoreMesh(core_axis_name="c", subcore_axis_name="s")
_params = pltpu.CompilerParams(needs_layout_passes=False)


def _wid():
    return lax.axis_index("s") * NC + lax.axis_index("c")


def _assign(w):
    start = 78 * w + jnp.minimum(w, 4)
    n = jnp.where(w < 4, 79, 78)
    return start, n


def _bytesums(v):
    u = v + (v << 8)
    return u + (u << 16)


@functools.partial(
    pl.kernel,
    out_type=jax.ShapeDtypeStruct((NW * CROW,), jnp.int32),
    mesh=_mesh,
    compiler_params=_params,
    scratch_types=[
        pltpu.VMEM((CHUNK,), jnp.float32),
        pltpu.VMEM((CHUNK,), jnp.float32),
        pltpu.VMEM((CROW,), jnp.int32),
        pltpu.SemaphoreType.DMA,
        pltpu.SemaphoreType.DMA,
    ],
)
def _count_kernel(mk_hbm, counts_hbm, mb0, mb1, cbuf, sm0, sm1):
    w = _wid()
    start_c, n_w = _assign(w)
    mbufs = (mb0, mb1)
    sems = (sm0, sm1)
    zeros = jnp.zeros((16,), jnp.int32)
    lane = lax.iota(jnp.int32, 16)
    m0 = lane == 0
    for j in range(CROW // 16):
        cbuf[pl.ds(j * 16, 16)] = zeros

    def mask_copy(i, u):
        return pltpu.make_async_copy(
            mk_hbm.at[pl.ds((start_c + i) * CHUNK, CHUNK)], mbufs[u], sems[u])

    mask_copy(0, 0).start()

    def pair_body(p, _):
        for u in range(2):
            i = 2 * p + u

            @pl.when(i < n_w)
            def _():
                mask_copy(i, u).wait()

                @pl.when(i + 1 < n_w)
                def _():
                    mask_copy(i + 1, 1 - u).start()

                @plsc.parallel_loop(0, WGROUPS,
                                    carry=jnp.zeros((16,), jnp.float32))
                def acc(g, a):
                    m = mbufs[u]
                    v0 = m[pl.ds(g * 64, 16)]
                    v1 = m[pl.ds(g * 64 + 16, 16)]
                    v2 = m[pl.ds(g * 64 + 32, 16)]
                    v3 = m[pl.ds(g * 64 + 48, 16)]
                    return a + ((v0 + v1) + (v2 + v3))

                cnt = jnp.sum(acc).astype(jnp.int32)
                plsc.store_scatter(cbuf, [zeros + i], zeros + cnt, mask=m0)

        return 0

    lax.fori_loop(0, (MAXCH + 1) // 2, pair_body, 0)
    pltpu.sync_copy(cbuf, counts_hbm.at[pl.ds(w * CROW, CROW)])


@functools.partial(
    pl.kernel,
    out_type=jax.ShapeDtypeStruct((N,), jnp.float32),
    mesh=_mesh,
    compiler_params=_params,
    scratch_types=(
        [pltpu.VMEM((CHUNK,), jnp.float32) for _ in range(NSLOT)]
        + [pltpu.VMEM((CHUNK,), jnp.float32) for _ in range(NSLOT)]
        + [pltpu.VMEM((SPAD,), jnp.float32) for _ in range(NSLOT)]
        + [
            pltpu.VMEM((NW * CROW,), jnp.int32),
            pltpu.VMEM((96,), jnp.int32),
        ]
        + [pltpu.SemaphoreType.DMA for _ in range(3 * NSLOT)]
    ),
)
def _scatter_kernel(mf_hbm, x_hbm, src_hbm, counts_hbm, out_hbm,
                    mb0, mb1, mb2, ob0, ob1, ob2, sb0, sb1, sb2,
                    cbuf, bbuf,
                    si0, si1, si2, sx0, sx1, sx2, so0, so1, so2):
    mbufs = (mb0, mb1, mb2)
    obufs = (ob0, ob1, ob2)
    sbufs = (sb0, sb1, sb2)
    sem_in = (si0, si1, si2)
    sem_x = (sx0, sx1, sx2)
    sem_out = (so0, so1, so2)
    w = _wid()
    start_c, n_w = _assign(w)
    zeros = jnp.zeros((16,), jnp.int32)
    lane = lax.iota(jnp.int32, 16)

    pltpu.sync_copy(counts_hbm, cbuf)

    def sb_body(j, acc):
        return acc + cbuf[pl.ds(j * 16, 16)]

    shard_base = jnp.sum(
        lax.fori_loop(0, w * (CROW // 16), sb_body, zeros))

    def bb_body(j, carry_s):
        cvec = cbuf[pl.ds((w * (CROW // 16) + j) * 16, 16)]
        bbuf[pl.ds(j * 16, 16)] = plsc.cumsum(cvec) - cvec + carry_s
        return carry_s + jnp.sum(cvec)

    lax.fori_loop(0, CROW // 16, bb_body, shard_base)

    def chunk_base(i):
        return bbuf[pl.ds(i, 16)][0]

    def win_base(i):
        return pl.multiple_of(
            jnp.minimum(chunk_base(i) & -8, N - SRCW), 8)

    def in_copies(i, u):
        start = (start_c + i) * CHUNK
        m = pltpu.make_async_copy(
            mf_hbm.at[pl.ds(start, CHUNK)], mbufs[u], sem_in[u])
        s = pltpu.make_async_copy(
            src_hbm.at[pl.ds(win_base(i), SRCW)],
            sbufs[u].at[pl.ds(0, SRCW)], sem_in[u])
        xc = pltpu.make_async_copy(
            x_hbm.at[pl.ds(start, CHUNK)], obufs[u], sem_x[u])
        return m, s, xc

    def out_copy(i, u):
        start = (start_c + i) * CHUNK
        return pltpu.make_async_copy(
            obufs[u], out_hbm.at[pl.ds(start, CHUNK)], sem_out[u])

    def do_chunk(i, u):
        mbuf, obuf, sbuf = mbufs[u], obufs[u], sbufs[u]
        for c in in_copies(i, u):
            c.wait()

        nu = (u + 2) % NSLOT

        @pl.when(i + 2 < n_w)
        def _():
            @pl.when(i >= 1)
            def _():
                out_copy(i - 1, nu).wait()

            for c in in_copies(i + 2, nu):
                c.start()

        base_s = chunk_base(i)
        off0 = zeros + (base_s - win_base(i))

        @plsc.parallel_loop(0, GROUPS, carry=off0)
        def _(g, off):
            mk = mbuf[pl.ds(g * 16, 16)] != 0.0
            vals = plsc.load_expanded(sbuf.at[pl.ds(off[0], 16)], mask=mk)
            plsc.store_scatter(obuf, [lane + g * 16], vals, mask=mk)
            return off + plsc.all_reduce_population_count(mk)

        out_copy(i, u).start()

    for c in in_copies(0, 0) + in_copies(1, 1):
        c.start()

    def triple_body(p, _):
        for u in range(NSLOT):
            i = 3 * p + u

            @pl.when(i < n_w)
            def _():
                do_chunk(i, u)

        return 0

    lax.fori_loop(0, (MAXCH + NSLOT - 1) // NSLOT, triple_body, 0)
    for u in range(NSLOT):
        out_copy(0, u).wait()


def kernel(x, mask, src):
    xf = x.reshape(-1)
    sf = src.reshape(-1)
    mf = mask.astype(jnp.float32).reshape(-1)
    counts = _count_kernel(mf)
    outf = _scatter_kernel(mf, xf, sf, counts)
    return outf.reshape(x.shape)


# speedup vs baseline: 23.5524x; 1.7796x over previous
import functools

import jax
import jax.numpy as jnp
from jax import lax
from jax.experimental import pallas as pl
from jax.experimental.pallas import tpu as pltpu
from jax.experimental.pallas import tpu_sc as plsc

M_ROWS = 250000
D = 128
N = M_ROWS * D
NC = 2
NS = 16
NW = NC * NS
CHUNK = 12800
NCHUNK_G = N // CHUNK
GROUPS = CHUNK // 16
WGROUPS = CHUNK // 64
MAXCH = 79
SRCW = CHUNK + 8
SPAD = SRCW + 16
CROW = 80
NSLOT = 3

_mesh = plsc.VectorSubcoreMesh(core_axis_name="c", subcore_axis_name="s")
_params = pltpu.CompilerParams(needs_layout_passes=False)


def _wid():
    return lax.axis_index("s") * NC + lax.axis_index("c")


def _assign(w):
    start = 78 * w + jnp.minimum(w, 4)
    n = jnp.where(w < 4, 79, 78)
    return start, n


def _bytesums(v):
    u = v + (v << 8)
    return u + (u << 16)


@functools.partial(
    pl.kernel,
    out_type=jax.ShapeDtypeStruct((NW * CROW,), jnp.int32),
    mesh=_mesh,
    compiler_params=_params,
    scratch_types=[
        pltpu.VMEM((CHUNK,), jnp.float32),
        pltpu.VMEM((CHUNK,), jnp.float32),
        pltpu.VMEM((CROW,), jnp.int32),
        pltpu.SemaphoreType.DMA,
        pltpu.SemaphoreType.DMA,
    ],
)
def _count_kernel(mk_hbm, counts_hbm, mb0, mb1, cbuf, sm0, sm1):
    w = _wid()
    start_c, n_w = _assign(w)
    mbufs = (mb0, mb1)
    sems = (sm0, sm1)
    zeros = jnp.zeros((16,), jnp.int32)
    lane = lax.iota(jnp.int32, 16)
    m0 = lane == 0
    for j in range(CROW // 16):
        cbuf[pl.ds(j * 16, 16)] = zeros

    def mask_copy(i, u):
        return pltpu.make_async_copy(
            mk_hbm.at[pl.ds((start_c + i) * CHUNK, CHUNK)], mbufs[u], sems[u])

    mask_copy(0, 0).start()

    def pair_body(p, _):
        for u in range(2):
            i = 2 * p + u

            @pl.when(i < n_w)
            def _():
                mask_copy(i, u).wait()

                @pl.when(i + 1 < n_w)
                def _():
                    mask_copy(i + 1, 1 - u).start()

                @plsc.parallel_loop(0, WGROUPS, unroll=2,
                                    carry=jnp.zeros((16,), jnp.float32))
                def acc(g, a):
                    m = mbufs[u]
                    v0 = m[pl.ds(g * 64, 16)]
                    v1 = m[pl.ds(g * 64 + 16, 16)]
                    v2 = m[pl.ds(g * 64 + 32, 16)]
                    v3 = m[pl.ds(g * 64 + 48, 16)]
                    return a + ((v0 + v1) + (v2 + v3))

                cnt = jnp.sum(acc).astype(jnp.int32)
                plsc.store_scatter(cbuf, [zeros + i], zeros + cnt, mask=m0)

        return 0

    lax.fori_loop(0, (MAXCH + 1) // 2, pair_body, 0)
    pltpu.sync_copy(cbuf, counts_hbm.at[pl.ds(w * CROW, CROW)])


@functools.partial(
    pl.kernel,
    out_type=jax.ShapeDtypeStruct((N,), jnp.float32),
    mesh=_mesh,
    compiler_params=_params,
    scratch_types=(
        [pltpu.VMEM((CHUNK,), jnp.float32) for _ in range(NSLOT)]
        + [pltpu.VMEM((CHUNK,), jnp.float32) for _ in range(NSLOT)]
        + [pltpu.VMEM((SPAD,), jnp.float32) for _ in range(NSLOT)]
        + [
            pltpu.VMEM((NW * CROW,), jnp.int32),
            pltpu.VMEM((96,), jnp.int32),
        ]
        + [pltpu.SemaphoreType.DMA for _ in range(3 * NSLOT)]
    ),
)
def _scatter_kernel(mf_hbm, x_hbm, src_hbm, counts_hbm, out_hbm,
                    mb0, mb1, mb2, ob0, ob1, ob2, sb0, sb1, sb2,
                    cbuf, bbuf,
                    si0, si1, si2, sx0, sx1, sx2, so0, so1, so2):
    mbufs = (mb0, mb1, mb2)
    obufs = (ob0, ob1, ob2)
    sbufs = (sb0, sb1, sb2)
    sem_in = (si0, si1, si2)
    sem_x = (sx0, sx1, sx2)
    sem_out = (so0, so1, so2)
    w = _wid()
    start_c, n_w = _assign(w)
    zeros = jnp.zeros((16,), jnp.int32)
    lane = lax.iota(jnp.int32, 16)

    pltpu.sync_copy(counts_hbm, cbuf)

    def sb_body(j, acc):
        return acc + cbuf[pl.ds(j * 16, 16)]

    shard_base = jnp.sum(
        lax.fori_loop(0, w * (CROW // 16), sb_body, zeros))

    def bb_body(j, carry_s):
        cvec = cbuf[pl.ds((w * (CROW // 16) + j) * 16, 16)]
        bbuf[pl.ds(j * 16, 16)] = plsc.cumsum(cvec) - cvec + carry_s
        return carry_s + jnp.sum(cvec)

    lax.fori_loop(0, CROW // 16, bb_body, shard_base)

    def chunk_base(i):
        return bbuf[pl.ds(i, 16)][0]

    def win_base(i):
        return pl.multiple_of(
            jnp.minimum(chunk_base(i) & -8, N - SRCW), 8)

    def in_copies(i, u):
        start = (start_c + i) * CHUNK
        m = pltpu.make_async_copy(
            mf_hbm.at[pl.ds(start, CHUNK)], mbufs[u], sem_in[u])
        s = pltpu.make_async_copy(
            src_hbm.at[pl.ds(win_base(i), SRCW)],
            sbufs[u].at[pl.ds(0, SRCW)], sem_in[u])
        xc = pltpu.make_async_copy(
            x_hbm.at[pl.ds(start, CHUNK)], obufs[u], sem_x[u])
        return m, s, xc

    def out_copy(i, u):
        start = (start_c + i) * CHUNK
        return pltpu.make_async_copy(
            obufs[u], out_hbm.at[pl.ds(start, CHUNK)], sem_out[u])

    def do_chunk(i, u):
        mbuf, obuf, sbuf = mbufs[u], obufs[u], sbufs[u]
        for c in in_copies(i, u):
            c.wait()

        nu = (u + 2) % NSLOT

        @pl.when(i + 2 < n_w)
        def _():
            @pl.when(i >= 1)
            def _():
                out_copy(i - 1, nu).wait()

            for c in in_copies(i + 2, nu):
                c.start()

        base_s = chunk_base(i)
        off0 = zeros + (base_s - win_base(i))

        @plsc.parallel_loop(0, GROUPS, unroll=4, carry=off0)
        def _(g, off):
            mk = mbuf[pl.ds(g * 16, 16)] != 0.0
            vals = plsc.load_expanded(sbuf.at[pl.ds(off[0], 16)], mask=mk)
            plsc.store_scatter(obuf, [lane + g * 16], vals, mask=mk)
            return off + plsc.all_reduce_population_count(mk)

        out_copy(i, u).start()

    for c in in_copies(0, 0) + in_copies(1, 1):
        c.start()

    def triple_body(p, _):
        for u in range(NSLOT):
            i = 3 * p + u

            @pl.when(i < n_w)
            def _():
                do_chunk(i, u)

        return 0

    lax.fori_loop(0, (MAXCH + NSLOT - 1) // NSLOT, triple_body, 0)
    for u in range(NSLOT):
        out_copy(0, u).wait()


def kernel(x, mask, src):
    xf = x.reshape(-1)
    sf = src.reshape(-1)
    mf = mask.astype(jnp.float32).reshape(-1)
    counts = _count_kernel(mf)
    outf = _scatter_kernel(mf, xf, sf, counts)
    return outf.reshape(x.shape)
